# baseline (speedup 1.0000x reference)
import functools

import jax
import jax.numpy as jnp
from jax import lax
from jax.experimental import pallas as pl
from jax.experimental.pallas import tpu as pltpu
from jax.experimental.pallas import tpu_sc as plsc

H = 24
W = 24
HW = H * W
NPLANES = 64 * 384
NTILES = 32
PLANES_PER_TILE = NPLANES // NTILES
CHUNK = 64
NCHUNKS = PLANES_PER_TILE // CHUNK
NV = HW // 16
HALF = 3
BIG = 1 << 20


def _tile_body(x_hbm, t_hbm, o_hbm, tbuf, buf, sem):
    del sem
    wid = lax.axis_index("s") * 2 + lax.axis_index("c")
    tile_base = wid * PLANES_PER_TILE
    pltpu.sync_copy(t_hbm.at[pl.ds(tile_base * 1, PLANES_PER_TILE)], tbuf)

    lane = lax.iota(jnp.int32, 16)
    krs, kcs = [], []
    for v in range(3):
        k = lane + v * 16
        kr = k // 6
        krs.append(kr)
        kcs.append(k - kr * 6)

    def plane_body(ci, p):
        idx = ci * CHUNK + p
        tv = tbuf[pl.ds((idx // 16) * 16, 16)]
        t = jnp.sum(jnp.where(lane == idx % 16, tv, 0.0))
        base = p * HW

        @pl.when(t != 0.0)
        def _():
            m = [buf[pl.ds(base + a * 16, 16)] for a in range(4)]
            cidx = [jnp.full((16,), a, jnp.int32) for a in range(4)]
            for c in range(4, NV):
                a = c % 4
                v = buf[pl.ds(base + c * 16, 16)]
                gt = v > m[a]
                m[a] = jnp.where(gt, v, m[a])
                cidx[a] = jnp.where(gt, jnp.int32(c), cidx[a])
            j = [cidx[a] * 16 + lane for a in range(4)]

            def comb(m1, j1, m2, j2):
                take2 = (m2 > m1) | ((m2 == m1) & (j2 < j1))
                return jnp.where(take2, m2, m1), jnp.where(take2, j2, j1)

            ma, ja = comb(m[0], j[0], m[1], j[1])
            mb, jb = comb(m[2], j[2], m[3], j[3])
            mf, jf = comb(ma, ja, mb, jb)
            gmax = jnp.max(mf)
            jm = jnp.min(jnp.where(mf == gmax, jf, BIG))

            hh = jm // W
            ww = jm % W
            h1 = jnp.clip(hh - HALF, 0, H - 1)
            h2 = jnp.clip(hh + HALF, 0, H - 1)
            w1 = jnp.clip(ww - HALF, 0, W - 1)
            w2 = jnp.clip(ww + HALF, 0, W - 1)
            dh = h2 - h1
            dw = w2 - w1
            area = jnp.full((16,), dh * dw, jnp.int32).astype(jnp.float32)
            lam = jnp.float32(HW) / (jnp.float32(HW) - area)

            for c in range(NV):
                sl = pl.ds(base + c * 16, 16)
                buf[sl] = buf[sl] * lam

            zero = jnp.zeros((16,), jnp.float32)
            for v in range(3):
                bidx = base + (h1 + krs[v]) * W + (w1 + kcs[v])
                msk = (krs[v] < dh) & (kcs[v] < dw)
                plsc.store_scatter(buf, [bidx], zero, mask=msk)

    def chunk_body(ci, carry):
        base_el = (tile_base + ci * CHUNK) * HW
        pltpu.sync_copy(x_hbm.at[pl.ds(base_el, CHUNK * HW)], buf)
        lax.fori_loop(0, CHUNK, lambda p, c: (plane_body(ci, p), c)[1], 0,
                      unroll=False)
        pltpu.sync_copy(buf, o_hbm.at[pl.ds(base_el, CHUNK * HW)])
        return carry

    lax.fori_loop(0, NCHUNKS, chunk_body, 0, unroll=False)


@jax.jit
def kernel(x, T):
    batch, channel, h, w = x.shape
    xf = x.reshape(-1)
    tf = T.reshape(-1)
    mesh = plsc.VectorSubcoreMesh(core_axis_name="c", subcore_axis_name="s")
    run = pl.kernel(
        _tile_body,
        out_type=jax.ShapeDtypeStruct((NPLANES * HW,), jnp.float32),
        mesh=mesh,
        scratch_types=[
            pltpu.VMEM((PLANES_PER_TILE,), jnp.float32),
            pltpu.VMEM((CHUNK * HW,), jnp.float32),
            pltpu.SemaphoreType.DMA,
        ],
        compiler_params=pltpu.CompilerParams(needs_layout_passes=False),
    )
    out = run(xf, tf)
    return out.reshape(batch, channel, h, w)
